# baseline (speedup 1.0000x reference)
import functools

import jax
import jax.numpy as jnp
from jax import lax
from jax.experimental import pallas as pl
from jax.experimental.pallas import tpu as pltpu
from jax.experimental.pallas import tpu_sc as plsc

DIM = 64
B = 16384
L = 20
NW = 32
BROW_W = B // NW
CHB = 32
NITER = BROW_W // CHB

_mesh = plsc.VectorSubcoreMesh(core_axis_name="c", subcore_axis_name="s")


@functools.partial(
    pl.kernel,
    mesh=_mesh,
    out_type=jax.ShapeDtypeStruct((B, L, DIM), jnp.float32),
    scratch_types=[
        pltpu.VMEM((BROW_W, L), jnp.int32),
        pltpu.VMEM((CHB, L, DIM), jnp.float32),
        pltpu.VMEM((CHB, L, DIM), jnp.float32),
        pltpu.SemaphoreType.DMA,
        pltpu.SemaphoreType.DMA,
        pltpu.SemaphoreType.DMA,
        pltpu.SemaphoreType.DMA,
    ],
    compiler_params=pltpu.CompilerParams(use_tc_tiling_on_sc=False),
)
def _emb_lookup(x_hbm, table_hbm, out_hbm, idx_v, rows0, rows1,
                gsem0, gsem1, ssem0, ssem1):
    wid = lax.axis_index("s") * 2 + lax.axis_index("c")
    brow_base = wid * BROW_W

    pltpu.sync_copy(x_hbm.at[pl.ds(brow_base, BROW_W)], idx_v)

    def fire(g, rows_buf, sem):
        def issue(r, carry):
            pltpu.async_copy(
                table_hbm.at[idx_v.at[g * CHB + r]], rows_buf.at[r], sem)
            return carry
        lax.fori_loop(0, CHB, issue, 0)

    def wait_gather(g, rows_buf, sem):
        pltpu.make_async_copy(
            out_hbm.at[pl.ds(brow_base + g * CHB, CHB)], rows_buf, sem,
        ).wait()

    def store(g, rows_buf, sem):
        pltpu.async_copy(
            rows_buf, out_hbm.at[pl.ds(brow_base + g * CHB, CHB)], sem)

    def wait_store(g, rows_buf, sem):
        pltpu.make_async_copy(
            rows_buf, out_hbm.at[pl.ds(brow_base + g * CHB, CHB)], sem,
        ).wait()

    fire(0, rows0, gsem0)

    def body(h, carry):
        g0 = 2 * h
        fire(g0 + 1, rows1, gsem1)
        wait_gather(g0, rows0, gsem0)
        store(g0, rows0, ssem0)
        wait_gather(g0 + 1, rows1, gsem1)
        store(g0 + 1, rows1, ssem1)
        wait_store(g0, rows0, ssem0)
        fire(g0 + 2, rows0, gsem0)
        wait_store(g0 + 1, rows1, ssem1)
        return carry

    lax.fori_loop(0, NITER // 2 - 1, body, 0)

    glast = NITER - 2
    fire(glast + 1, rows1, gsem1)
    wait_gather(glast, rows0, gsem0)
    pltpu.sync_copy(rows0, out_hbm.at[pl.ds(brow_base + glast * CHB, CHB)])
    wait_gather(glast + 1, rows1, gsem1)
    pltpu.sync_copy(rows1,
                    out_hbm.at[pl.ds(brow_base + (glast + 1) * CHB, CHB)])


def kernel(x, table):
    return _emb_lookup(x.astype(jnp.int32), table)
